# baseline (speedup 1.0000x reference)
import functools

import jax
import jax.numpy as jnp
from jax import lax
from jax.experimental import pallas as pl
from jax.experimental.pallas import tpu as pltpu
from jax.experimental.pallas import tpu_sc as plsc

N_TOK = 8192
HID = 4096
LANES = 16
NC = 2
NS = 16
NW = NC * NS
TPW = N_TOK // NW
C = 4
NCHUNK = TPW // C
SPLIT = 2
VPT = HID // LANES


def _build():
    mesh = plsc.VectorSubcoreMesh(core_axis_name="c", subcore_axis_name="s")

    @functools.partial(
        pl.kernel,
        out_type=jax.ShapeDtypeStruct((N_TOK, HID), jnp.float32),
        mesh=mesh,
        compiler_params=pltpu.CompilerParams(use_tc_tiling_on_sc=False),
        scratch_types=[
            pltpu.VMEM((NCHUNK, 2 * C // SPLIT, SPLIT), jnp.int32),
            pltpu.VMEM((TPW + LANES,), jnp.float32),
            pltpu.VMEM((TPW + LANES,), jnp.float32),
            *[pltpu.VMEM((SPLIT, HID), jnp.float32)
              for _ in range(2 * (2 * C // SPLIT))],
            pltpu.VMEM((C, HID), jnp.float32),
            pltpu.VMEM((C, HID), jnp.float32),
            pltpu.SemaphoreType.DMA,
            pltpu.SemaphoreType.DMA,
            pltpu.SemaphoreType.DMA,
            pltpu.SemaphoreType.DMA,
        ],
    )
    def k(moe_hbm, idx_hbm, s0_hbm, s1_hbm, out_hbm,
          idx_v, w0_v, w1_v, *rest):
        nstream = 2 * C // SPLIT
        gbufs = rest[:2 * nstream]
        o0, o1 = rest[2 * nstream:2 * nstream + 2]
        sem_i0, sem_i1, sem_o0, sem_o1 = rest[2 * nstream + 2:]
        wid = lax.axis_index("s") * NC + lax.axis_index("c")
        base = wid * TPW
        pltpu.sync_copy(idx_hbm.at[wid], idx_v)
        pltpu.sync_copy(s0_hbm.at[wid], w0_v.at[pl.ds(0, TPW)])
        pltpu.sync_copy(s1_hbm.at[wid], w1_v.at[pl.ds(0, TPW)])

        bufs_g = (gbufs[:nstream], gbufs[nstream:])
        bufs_o = (o0, o1)
        sems_i = (sem_i0, sem_i1)
        sems_o = (sem_o0, sem_o1)

        def norm_body(i, carry):
            sl = pl.ds(i * LANES, LANES)
            a = w0_v[sl]
            b = w1_v[sl]
            t = a + b
            w0_v[sl] = a / t
            w1_v[sl] = b / t
            return carry

        lax.fori_loop(0, TPW // LANES, norm_body, 0)

        def start_gather(c, p):
            for j in range(nstream):
                pltpu.async_copy(moe_hbm.at[idx_v.at[c, j]],
                                 bufs_g[p][j], sems_i[p])

        def wait_gather(p):
            for j in range(nstream):
                pltpu.make_async_copy(moe_hbm.at[idx_v.at[0, j]],
                                      bufs_g[p][j], sems_i[p]).wait()

        def wait_out(p):
            pltpu.make_async_copy(bufs_o[p], out_hbm.at[pl.ds(base, C)], sems_o[p]).wait()

        start_gather(0, 0)

        def outer(g, carry):
            for p in range(2):
                c = g * 2 + p
                if p == 0:
                    start_gather(c + 1, 1)
                else:
                    @pl.when(g + 1 < NCHUNK // 2)
                    def _():
                        start_gather(c + 1, 0)
                wait_gather(p)

                @pl.when(g >= 1)
                def _():
                    wait_out(p)

                w0c = w0_v[pl.ds(c * C, LANES)]
                w1c = w1_v[pl.ds(c * C, LANES)]
                w0s = [jnp.full((LANES,), w0c[t]) for t in range(C)]
                w1s = [jnp.full((LANES,), w1c[t]) for t in range(C)]
                buf_g = bufs_g[p]
                buf_o = bufs_o[p]

                nsa = C // SPLIT

                def vec_body(v, carry2):
                    sl = pl.ds(v * LANES, LANES)
                    for t in range(C):
                        a = buf_g[t // SPLIT][t % SPLIT, sl]
                        b = buf_g[nsa + t // SPLIT][t % SPLIT, sl]
                        buf_o[t, sl] = a * w0s[t] + b * w1s[t]
                    return carry2

                lax.fori_loop(0, VPT, vec_body, 0, unroll=2)
                pltpu.async_copy(buf_o, out_hbm.at[pl.ds(base + c * C, C)], sems_o[p])
            return carry

        lax.fori_loop(0, NCHUNK // 2, outer, 0)
        wait_out(0)
        wait_out(1)

    return k


_sc_combine = _build()


def kernel(moe_output, scores, mapped_slots, expert_counts):
    del expert_counts
    idx = (mapped_slots.reshape(NW, NCHUNK, C, 2)
           .transpose(0, 1, 3, 2)
           .reshape(NW, NCHUNK, 2 * C // SPLIT, SPLIT))
    s0 = scores[:, 0].reshape(NW, TPW)
    s1 = scores[:, 1].reshape(NW, TPW)
    return _sc_combine(moe_output, idx, s0, s1)


# speedup vs baseline: 1.0046x; 1.0046x over previous
import functools

import jax
import jax.numpy as jnp
from jax import lax
from jax.experimental import pallas as pl
from jax.experimental.pallas import tpu as pltpu
from jax.experimental.pallas import tpu_sc as plsc

N_TOK = 8192
HID = 4096
LANES = 16
NC = 2
NS = 16
NW = NC * NS
TPW = N_TOK // NW
C = 4
NCHUNK = TPW // C
SPLIT = 4
VPT = HID // LANES


def _build():
    mesh = plsc.VectorSubcoreMesh(core_axis_name="c", subcore_axis_name="s")

    @functools.partial(
        pl.kernel,
        out_type=jax.ShapeDtypeStruct((N_TOK, HID), jnp.float32),
        mesh=mesh,
        compiler_params=pltpu.CompilerParams(use_tc_tiling_on_sc=False),
        scratch_types=[
            pltpu.VMEM((NCHUNK, 2 * C // SPLIT, SPLIT), jnp.int32),
            pltpu.VMEM((TPW + LANES,), jnp.float32),
            pltpu.VMEM((TPW + LANES,), jnp.float32),
            *[pltpu.VMEM((SPLIT, HID), jnp.float32)
              for _ in range(2 * (2 * C // SPLIT))],
            pltpu.VMEM((C, HID), jnp.float32),
            pltpu.VMEM((C, HID), jnp.float32),
            pltpu.SemaphoreType.DMA,
            pltpu.SemaphoreType.DMA,
            pltpu.SemaphoreType.DMA,
            pltpu.SemaphoreType.DMA,
        ],
    )
    def k(moe_hbm, idx_hbm, s0_hbm, s1_hbm, out_hbm,
          idx_v, w0_v, w1_v, *rest):
        nstream = 2 * C // SPLIT
        gbufs = rest[:2 * nstream]
        o0, o1 = rest[2 * nstream:2 * nstream + 2]
        sem_i0, sem_i1, sem_o0, sem_o1 = rest[2 * nstream + 2:]
        wid = lax.axis_index("s") * NC + lax.axis_index("c")
        base = wid * TPW
        pltpu.sync_copy(idx_hbm.at[wid], idx_v)
        pltpu.sync_copy(s0_hbm.at[wid], w0_v.at[pl.ds(0, TPW)])
        pltpu.sync_copy(s1_hbm.at[wid], w1_v.at[pl.ds(0, TPW)])

        bufs_g = (gbufs[:nstream], gbufs[nstream:])
        bufs_o = (o0, o1)
        sems_i = (sem_i0, sem_i1)
        sems_o = (sem_o0, sem_o1)

        def norm_body(i, carry):
            sl = pl.ds(i * LANES, LANES)
            a = w0_v[sl]
            b = w1_v[sl]
            t = a + b
            w0_v[sl] = a / t
            w1_v[sl] = b / t
            return carry

        lax.fori_loop(0, TPW // LANES, norm_body, 0)

        def start_gather(c, p):
            for j in range(nstream):
                pltpu.async_copy(moe_hbm.at[idx_v.at[c, j]],
                                 bufs_g[p][j], sems_i[p])

        def wait_gather(p):
            for j in range(nstream):
                pltpu.make_async_copy(moe_hbm.at[idx_v.at[0, j]],
                                      bufs_g[p][j], sems_i[p]).wait()

        def wait_out(p):
            pltpu.make_async_copy(bufs_o[p], out_hbm.at[pl.ds(base, C)], sems_o[p]).wait()

        start_gather(0, 0)

        def outer(g, carry):
            for p in range(2):
                c = g * 2 + p
                if p == 0:
                    start_gather(c + 1, 1)
                else:
                    @pl.when(g + 1 < NCHUNK // 2)
                    def _():
                        start_gather(c + 1, 0)
                wait_gather(p)

                @pl.when(g >= 1)
                def _():
                    wait_out(p)

                w0c = w0_v[pl.ds(c * C, LANES)]
                w1c = w1_v[pl.ds(c * C, LANES)]
                w0s = [jnp.full((LANES,), w0c[t]) for t in range(C)]
                w1s = [jnp.full((LANES,), w1c[t]) for t in range(C)]
                buf_g = bufs_g[p]
                buf_o = bufs_o[p]

                nsa = C // SPLIT

                def vec_body(v, carry2):
                    sl = pl.ds(v * LANES, LANES)
                    for t in range(C):
                        a = buf_g[t // SPLIT][t % SPLIT, sl]
                        b = buf_g[nsa + t // SPLIT][t % SPLIT, sl]
                        buf_o[t, sl] = a * w0s[t] + b * w1s[t]
                    return carry2

                lax.fori_loop(0, VPT, vec_body, 0, unroll=2)
                pltpu.async_copy(buf_o, out_hbm.at[pl.ds(base + c * C, C)], sems_o[p])
            return carry

        lax.fori_loop(0, NCHUNK // 2, outer, 0)
        wait_out(0)
        wait_out(1)

    return k


_sc_combine = _build()


def kernel(moe_output, scores, mapped_slots, expert_counts):
    del expert_counts
    idx = (mapped_slots.reshape(NW, NCHUNK, C, 2)
           .transpose(0, 1, 3, 2)
           .reshape(NW, NCHUNK, 2 * C // SPLIT, SPLIT))
    s0 = scores[:, 0].reshape(NW, TPW)
    s1 = scores[:, 1].reshape(NW, TPW)
    return _sc_combine(moe_output, idx, s0, s1)


# speedup vs baseline: 2.4076x; 2.3966x over previous
import functools

import jax
import jax.numpy as jnp
from jax import lax
from jax.experimental import pallas as pl
from jax.experimental.pallas import tpu as pltpu
from jax.experimental.pallas import tpu_sc as plsc

N_TOK = 8192
HID = 4096
LANES = 16
NC = 2
NS = 16
NW = NC * NS
TPW = N_TOK // NW
C = 4
NCHUNK = TPW // C
NSTREAM = 2
DEPTH = 3
VPT = HID // LANES


def _build():
    mesh = plsc.VectorSubcoreMesh(core_axis_name="c", subcore_axis_name="s")

    @functools.partial(
        pl.kernel,
        out_type=jax.ShapeDtypeStruct((N_TOK, HID), jnp.float32),
        mesh=mesh,
        scratch_types=[
            pltpu.VMEM((NCHUNK, NSTREAM * C), jnp.int32),
            pltpu.VMEM((TPW + LANES,), jnp.float32),
            pltpu.VMEM((TPW + LANES,), jnp.float32),
            *[pltpu.VMEM((C, HID), jnp.float32)
              for _ in range(DEPTH * NSTREAM)],
            pltpu.VMEM((C, HID), jnp.float32),
            *[pltpu.SemaphoreType.DMA for _ in range(DEPTH + 1)],
        ],
    )
    def k(moe_hbm, idx_hbm, s0_hbm, s1_hbm, out_hbm, idx_v, w0_v, w1_v, *rest):
        gflat = rest[:DEPTH * NSTREAM]
        bufs_g = [gflat[p * NSTREAM:(p + 1) * NSTREAM] for p in range(DEPTH)]
        buf_o = rest[DEPTH * NSTREAM]
        sems_i = rest[DEPTH * NSTREAM + 1:DEPTH * NSTREAM + 1 + DEPTH]
        sem_o = rest[DEPTH * NSTREAM + 1 + DEPTH]

        wid = lax.axis_index("s") * NC + lax.axis_index("c")
        base = wid * TPW
        pltpu.sync_copy(idx_hbm.at[wid], idx_v)
        pltpu.sync_copy(s0_hbm.at[wid], w0_v.at[pl.ds(0, TPW)])
        pltpu.sync_copy(s1_hbm.at[wid], w1_v.at[pl.ds(0, TPW)])

        def norm_body(i, carry):
            sl = pl.ds(i * LANES, LANES)
            a = w0_v[sl]
            b = w1_v[sl]
            t = a + b
            w0_v[sl] = a / t
            w1_v[sl] = b / t
            return carry

        lax.fori_loop(0, TPW // LANES, norm_body, 0)

        def start_gather(c, p):
            for j in range(NSTREAM):
                pltpu.async_copy(moe_hbm.at[idx_v.at[c, pl.ds(j * C, C)]],
                                 bufs_g[p][j], sems_i[p])

        def wait_gather(p):
            for j in range(NSTREAM):
                pltpu.make_async_copy(moe_hbm.at[idx_v.at[0, pl.ds(j * C, C)]],
                                      bufs_g[p][j], sems_i[p]).wait()

        def wait_out():
            pltpu.make_async_copy(buf_o, out_hbm.at[pl.ds(base, C)],
                                  sem_o).wait()

        def compute(c, p):
            w0c = w0_v[pl.ds(c * C, LANES)]
            w1c = w1_v[pl.ds(c * C, LANES)]
            w0s = [jnp.full((LANES,), w0c[t]) for t in range(C)]
            w1s = [jnp.full((LANES,), w1c[t]) for t in range(C)]
            buf_a, buf_b = bufs_g[p]

            def vec_body(v, carry2):
                sl = pl.ds(v * LANES, LANES)
                for t in range(C):
                    buf_o[t, sl] = buf_a[t, sl] * w0s[t] + buf_b[t, sl] * w1s[t]
                return carry2

            lax.fori_loop(0, VPT, vec_body, 0, unroll=2)
            pltpu.async_copy(buf_o, out_hbm.at[pl.ds(base + c * C, C)], sem_o)

        for p in range(DEPTH):
            start_gather(p, p)

        n_main = (NCHUNK - 1) // DEPTH

        def outer(g, carry):
            for p in range(DEPTH):
                c = g * DEPTH + p
                wait_gather(p)

                @pl.when(c >= 1)
                def _():
                    wait_out()

                compute(c, p)

                @pl.when(c + DEPTH < NCHUNK)
                def _():
                    start_gather(c + DEPTH, p)
            return carry

        lax.fori_loop(0, n_main, outer, 0)
        for c in range(n_main * DEPTH, NCHUNK):
            p = c % DEPTH
            wait_gather(p)
            wait_out()
            compute(c, p)
        wait_out()

    return k


_sc_combine = _build()


def kernel(moe_output, scores, mapped_slots, expert_counts):
    del expert_counts
    idx = (mapped_slots.reshape(NW, NCHUNK, C, NSTREAM)
           .transpose(0, 1, 3, 2)
           .reshape(NW, NCHUNK, NSTREAM * C))
    s0 = scores[:, 0].reshape(NW, TPW)
    s1 = scores[:, 1].reshape(NW, TPW)
    return _sc_combine(moe_output, idx, s0, s1)


# speedup vs baseline: 2.4844x; 1.0319x over previous
import functools

import jax
import jax.numpy as jnp
from jax import lax
from jax.experimental import pallas as pl
from jax.experimental.pallas import tpu as pltpu
from jax.experimental.pallas import tpu_sc as plsc

N_TOK = 8192
HID = 4096
LANES = 16
NC = 2
NS = 16
NW = NC * NS
TPW = N_TOK // NW
C = 4
NCHUNK = TPW // C
HSPLIT = 2
HCOL = HID // HSPLIT
NSTREAM = 2 * HSPLIT
VPH = HCOL // LANES


def _build():
    mesh = plsc.VectorSubcoreMesh(core_axis_name="c", subcore_axis_name="s")

    @functools.partial(
        pl.kernel,
        out_type=jax.ShapeDtypeStruct((N_TOK, HID), jnp.float32),
        mesh=mesh,
        scratch_types=[
            pltpu.VMEM((NCHUNK, 16), jnp.int32),
            pltpu.VMEM((TPW + LANES,), jnp.float32),
            pltpu.VMEM((TPW + LANES,), jnp.float32),
            *[pltpu.VMEM((C, HCOL), jnp.float32)
              for _ in range(2 * NSTREAM)],
            pltpu.VMEM((C, HID), jnp.float32),
            pltpu.VMEM((C, HID), jnp.float32),
            pltpu.SemaphoreType.DMA,
            pltpu.SemaphoreType.DMA,
            pltpu.SemaphoreType.DMA,
            pltpu.SemaphoreType.DMA,
        ],
    )
    def k(moe_hbm, idx_hbm, s0_hbm, s1_hbm, out_hbm, idx_v, w0_v, w1_v, *rest):
        gflat = rest[:2 * NSTREAM]
        bufs_g = (gflat[:NSTREAM], gflat[NSTREAM:])
        bufs_o = rest[2 * NSTREAM:2 * NSTREAM + 2]
        sems_i = rest[2 * NSTREAM + 2:2 * NSTREAM + 4]
        sems_o = rest[2 * NSTREAM + 4:2 * NSTREAM + 6]

        wid = lax.axis_index("s") * NC + lax.axis_index("c")
        base = wid * TPW
        pltpu.sync_copy(idx_hbm.at[wid], idx_v)
        pltpu.sync_copy(s0_hbm.at[wid], w0_v.at[pl.ds(0, TPW)])
        pltpu.sync_copy(s1_hbm.at[wid], w1_v.at[pl.ds(0, TPW)])

        def norm_body(i, carry):
            sl = pl.ds(i * LANES, LANES)
            a = w0_v[sl]
            b = w1_v[sl]
            t = a + b
            w0_v[sl] = a / t
            w1_v[sl] = b / t
            return carry

        lax.fori_loop(0, TPW // LANES, norm_body, 0)

        def start_gather(c, p):
            for j in range(NSTREAM):
                ioff = (j // HSPLIT) * 8
                coff = (j % HSPLIT) * HCOL
                pltpu.async_copy(
                    moe_hbm.at[idx_v.at[c, pl.ds(ioff, C)], pl.ds(coff, HCOL)],
                    bufs_g[p][j], sems_i[p])

        def wait_gather(p):
            for j in range(NSTREAM):
                ioff = (j // HSPLIT) * 8
                coff = (j % HSPLIT) * HCOL
                pltpu.make_async_copy(
                    moe_hbm.at[idx_v.at[0, pl.ds(ioff, C)], pl.ds(coff, HCOL)],
                    bufs_g[p][j], sems_i[p]).wait()

        def wait_out(p):
            pltpu.make_async_copy(bufs_o[p], out_hbm.at[pl.ds(base, C)],
                                  sems_o[p]).wait()

        def compute(c, p):
            w0c = w0_v[pl.ds(c * C, LANES)]
            w1c = w1_v[pl.ds(c * C, LANES)]
            w0s = [jnp.full((LANES,), w0c[t]) for t in range(C)]
            w1s = [jnp.full((LANES,), w1c[t]) for t in range(C)]
            buf_o = bufs_o[p]
            for h in range(HSPLIT):
                buf_a = bufs_g[p][h]
                buf_b = bufs_g[p][HSPLIT + h]

                def vec_body(v, carry2):
                    sl = pl.ds(v * LANES, LANES)
                    osl = pl.ds(h * HCOL + v * LANES, LANES)
                    for t in range(C):
                        buf_o[t, osl] = (buf_a[t, sl] * w0s[t]
                                         + buf_b[t, sl] * w1s[t])
                    return carry2

                lax.fori_loop(0, VPH, vec_body, 0, unroll=2)
            pltpu.async_copy(buf_o, out_hbm.at[pl.ds(base + c * C, C)],
                             sems_o[p])

        start_gather(0, 0)
        start_gather(1, 1)

        def outer(g, carry):
            for p in range(2):
                c = g * 2 + p
                wait_gather(p)

                @pl.when(c >= 2)
                def _():
                    wait_out(p)

                compute(c, p)

                @pl.when(c + 2 < NCHUNK)
                def _():
                    start_gather(c + 2, p)
            return carry

        lax.fori_loop(0, NCHUNK // 2, outer, 0)
        wait_out(0)
        wait_out(1)

    return k


_sc_combine = _build()


def kernel(moe_output, scores, mapped_slots, expert_counts):
    del expert_counts
    ms = mapped_slots.reshape(NW, NCHUNK, C, 2)
    idx = jnp.zeros((NW, NCHUNK, 16), jnp.int32)
    idx = idx.at[:, :, 0:C].set(ms[..., 0])
    idx = idx.at[:, :, 8:8 + C].set(ms[..., 1])
    s0 = scores[:, 0].reshape(NW, TPW)
    s1 = scores[:, 1].reshape(NW, TPW)
    return _sc_combine(moe_output, idx, s0, s1)
